# baseline (speedup 1.0000x reference)
import functools

import jax
import jax.numpy as jnp
from jax import lax
from jax.experimental import pallas as pl
from jax.experimental.pallas import tpu as pltpu
from jax.experimental.pallas import tpu_sc as plsc

L = 16
NC, NS = 2, 16
NW = NC * NS


@functools.partial(jax.jit, static_argnames=("n", "m", "topk", "c_kv"))
def _gather_scale(kv16, idx_flat, w_flat, *, n, m, topk, c_kv):
    R = n * m * topk
    RPW = R // NW
    CH = 16
    NBUF = 4
    PF = 3
    NCH = RPW // CH
    G = c_kv // (2 * L)
    assert NCH % NBUF == 0 and PF < NBUF

    mesh = plsc.VectorSubcoreMesh(core_axis_name="c", subcore_axis_name="s")

    @functools.partial(
        pl.kernel,
        out_type=jax.ShapeDtypeStruct((R, c_kv), jnp.float32),
        mesh=mesh,
        scratch_types=[
            pltpu.VMEM((RPW,), jnp.int32),
            pltpu.VMEM((RPW,), jnp.float32),
            pltpu.VMEM((NBUF, CH, c_kv), jnp.float32),
            pltpu.VMEM_SHARED((NS, 2, CH, c_kv), jnp.float32),
        ]
        + [pltpu.SemaphoreType.DMA] * (NBUF + 2),
        compiler_params=pltpu.CompilerParams(needs_layout_passes=False),
    )
    def k(kv_hbm, idx_hbm, w_hbm, out_hbm, idx_v, w_v, buf_v, shm_v, *sems):
        gs, ss = sems[:NBUF], sems[NBUF:]
        cid = lax.axis_index("c")
        sid = lax.axis_index("s")
        wid = cid * NS + sid
        base = wid * RPW

        pltpu.sync_copy(idx_hbm.at[pl.ds(base, RPW)], idx_v)
        pltpu.sync_copy(w_hbm.at[pl.ds(base, RPW)], w_v)

        boff = (base // (m * topk)) * m

        @pl.loop(0, RPW // L)
        def _(i):
            idx_v[pl.ds(i * L, L)] = idx_v[pl.ds(i * L, L)] + boff

        def gather_start(c, b):
            pltpu.async_copy(
                kv_hbm.at[idx_v.at[pl.ds(c * CH, CH)]], buf_v.at[b], gs[b]
            )

        def gather_wait(b):
            pltpu.make_async_copy(
                kv_hbm.at[pl.ds(0, CH)], buf_v.at[b], gs[b]
            ).wait()

        def scatter_start(c, b, s):
            pltpu.sync_copy(buf_v.at[b], shm_v.at[sid].at[s])
            pltpu.async_copy(
                shm_v.at[sid].at[s], out_hbm.at[pl.ds(base + c * CH, CH)],
                ss[s],
            )

        def scatter_wait(s):
            pltpu.make_async_copy(
                shm_v.at[sid].at[s], out_hbm.at[pl.ds(base, CH)], ss[s]
            ).wait()

        for c in range(PF):
            gather_start(c, c % NBUF)

        @pl.loop(0, NCH, step=NBUF)
        def _(c0):
            for t in range(NBUF):
                c = c0 + t
                b = t
                cp = c + PF
                bp = (t + PF) % NBUF

                @pl.when(cp < NCH)
                def _():
                    gather_start(cp, bp)

                gather_wait(b)
                row0 = c * CH
                for j in range(CH):
                    wv = plsc.load_gather(
                        w_v, [jnp.full((L,), row0 + j, jnp.int32)]
                    )

                    @pl.loop(0, 2 * G, unroll=8)
                    def _(g):
                        buf_v[b, j, pl.ds(g * L, L)] = (
                            buf_v[b, j, pl.ds(g * L, L)] * wv
                        )

                @pl.when(c >= 2)
                def _():
                    scatter_wait(t % 2)

                scatter_start(c, b, t % 2)

        for s in range(2):
            scatter_wait(s)

    return k(kv16, idx_flat, w_flat)


def kernel(r_idx, r_weight, kv):
    n, m, c_kv = kv.shape
    topk = r_idx.shape[-1]
    R = n * m * topk
    kv16 = kv.reshape(n * m, c_kv)
    idx_flat = r_idx.reshape(R).astype(jnp.int32)
    w_flat = r_weight.reshape(R)
    out = _gather_scale(kv16, idx_flat, w_flat, n=n, m=m, topk=topk, c_kv=c_kv)
    return out.reshape(n, m, topk, c_kv)


# speedup vs baseline: 1.0107x; 1.0107x over previous
import functools

import jax
import jax.numpy as jnp
from jax import lax
from jax.experimental import pallas as pl
from jax.experimental.pallas import tpu as pltpu
from jax.experimental.pallas import tpu_sc as plsc

L = 16
NC, NS = 2, 16
NW = NC * NS


@functools.partial(jax.jit, static_argnames=("n", "m", "topk", "c_kv"))
def _gather_scale(kv16, idx_flat, w_flat, *, n, m, topk, c_kv):
    R = n * m * topk
    RPW = R // NW
    CH = 16
    NBUF = 4
    PF = 3
    NCH = RPW // CH
    G = c_kv // (2 * L)
    assert NCH % NBUF == 0 and PF < NBUF

    mesh = plsc.VectorSubcoreMesh(core_axis_name="c", subcore_axis_name="s")

    @functools.partial(
        pl.kernel,
        out_type=jax.ShapeDtypeStruct((R, c_kv), jnp.float32),
        mesh=mesh,
        scratch_types=[
            pltpu.VMEM((RPW,), jnp.int32),
            pltpu.VMEM((RPW,), jnp.float32),
            pltpu.VMEM((NBUF, CH, c_kv), jnp.float32),
        ]
        + [pltpu.SemaphoreType.DMA] * (2 * NBUF),
        compiler_params=pltpu.CompilerParams(needs_layout_passes=False),
    )
    def k(kv_hbm, idx_hbm, w_hbm, out_hbm, idx_v, w_v, buf_v, *sems):
        gs, ss = sems[:NBUF], sems[NBUF:]
        cid = lax.axis_index("c")
        sid = lax.axis_index("s")
        wid = cid * NS + sid
        base = wid * RPW

        pltpu.sync_copy(idx_hbm.at[pl.ds(base, RPW)], idx_v)
        pltpu.sync_copy(w_hbm.at[pl.ds(base, RPW)], w_v)

        boff = (base // (m * topk)) * m

        @pl.loop(0, RPW // L)
        def _(i):
            idx_v[pl.ds(i * L, L)] = idx_v[pl.ds(i * L, L)] + boff

        def gather_start(c, b):
            pltpu.async_copy(
                kv_hbm.at[idx_v.at[pl.ds(c * CH, CH)]], buf_v.at[b], gs[b]
            )

        def gather_wait(b):
            pltpu.make_async_copy(
                kv_hbm.at[pl.ds(0, CH)], buf_v.at[b], gs[b]
            ).wait()

        def scatter_start(c, b):
            pltpu.async_copy(
                buf_v.at[b], out_hbm.at[pl.ds(base + c * CH, CH)], ss[b]
            )

        def scatter_wait(b):
            pltpu.make_async_copy(
                buf_v.at[b], out_hbm.at[pl.ds(base, CH)], ss[b]
            ).wait()

        for c in range(PF):
            gather_start(c, c % NBUF)

        @pl.loop(0, NCH, step=NBUF)
        def _(c0):
            for t in range(NBUF):
                c = c0 + t
                b = t
                cp = c + PF
                bp = (t + PF) % NBUF

                @pl.when(cp < NCH)
                def _():
                    @pl.when(cp >= NBUF)
                    def _():
                        scatter_wait(bp)

                    gather_start(cp, bp)

                gather_wait(b)
                row0 = c * CH
                for j in range(CH):
                    wv = plsc.load_gather(
                        w_v, [jnp.full((L,), row0 + j, jnp.int32)]
                    )

                    @pl.loop(0, 2 * G, unroll=8)
                    def _(g):
                        buf_v[b, j, pl.ds(g * L, L)] = (
                            buf_v[b, j, pl.ds(g * L, L)] * wv
                        )

                scatter_start(c, b)

        for b in range(NBUF):
            scatter_wait(b)

    return k(kv16, idx_flat, w_flat)


def kernel(r_idx, r_weight, kv):
    n, m, c_kv = kv.shape
    topk = r_idx.shape[-1]
    R = n * m * topk
    kv16 = kv.reshape(n * m, c_kv)
    idx_flat = r_idx.reshape(R).astype(jnp.int32)
    w_flat = r_weight.reshape(R)
    out = _gather_scale(kv16, idx_flat, w_flat, n=n, m=m, topk=topk, c_kv=c_kv)
    return out.reshape(n, m, topk, c_kv)


# speedup vs baseline: 1.0109x; 1.0002x over previous
import functools

import jax
import jax.numpy as jnp
from jax import lax
from jax.experimental import pallas as pl
from jax.experimental.pallas import tpu as pltpu
from jax.experimental.pallas import tpu_sc as plsc

L = 16
NC, NS = 2, 16
NW = NC * NS


@functools.partial(jax.jit, static_argnames=("n", "m", "topk", "c_kv"))
def _gather_scale(kv_flat, idx_flat, w_flat, *, n, m, topk, c_kv):
    R = n * m * topk
    RPW = R // NW
    CH = 16
    NBUF = 4
    PF = 3
    NCH = RPW // CH
    VPR = c_kv // L
    assert NCH % NBUF == 0 and PF < NBUF

    mesh = plsc.VectorSubcoreMesh(core_axis_name="c", subcore_axis_name="s")

    @functools.partial(
        pl.kernel,
        out_type=jax.ShapeDtypeStruct((R, c_kv), jnp.float32),
        mesh=mesh,
        scratch_types=[
            pltpu.VMEM((RPW,), jnp.int32),
            pltpu.VMEM((RPW,), jnp.float32),
            pltpu.VMEM((NBUF, CH, c_kv), jnp.float32),
        ]
        + [pltpu.SemaphoreType.DMA] * (2 * NBUF),
        compiler_params=pltpu.CompilerParams(needs_layout_passes=False),
    )
    def k(kv_hbm, idx_hbm, w_hbm, out_hbm, idx_v, w_v, buf_v, *sems):
        gs, ss = sems[:NBUF], sems[NBUF:]
        cid = lax.axis_index("c")
        sid = lax.axis_index("s")
        wid = cid * NS + sid
        base = wid * RPW

        pltpu.sync_copy(idx_hbm.at[pl.ds(base, RPW)], idx_v)
        pltpu.sync_copy(w_hbm.at[pl.ds(base, RPW)], w_v)

        boff = (base // (m * topk)) * m

        @pl.loop(0, RPW // L)
        def _(i):
            idx_v[pl.ds(i * L, L)] = idx_v[pl.ds(i * L, L)] + boff

        def gather_start(c, b):
            pltpu.async_copy(
                kv_hbm.at[idx_v.at[pl.ds(c * CH, CH)]], buf_v.at[b], gs[b]
            )

        def gather_wait(b):
            pltpu.make_async_copy(
                kv_hbm.at[pl.ds(0, CH)], buf_v.at[b], gs[b]
            ).wait()

        def scatter_start(c, b):
            pltpu.async_copy(
                buf_v.at[b], out_hbm.at[pl.ds(base + c * CH, CH)], ss[b]
            )

        def scatter_wait(b):
            pltpu.make_async_copy(
                buf_v.at[b], out_hbm.at[pl.ds(base, CH)], ss[b]
            ).wait()

        for c in range(PF):
            gather_start(c, c % NBUF)

        @pl.loop(0, NCH, step=NBUF)
        def _(c0):
            for t in range(NBUF):
                c = c0 + t
                b = t
                cp = c + PF
                bp = (t + PF) % NBUF

                @pl.when(cp < NCH)
                def _():
                    @pl.when(cp >= NBUF)
                    def _():
                        scatter_wait(bp)

                    gather_start(cp, bp)

                gather_wait(b)
                row0 = c * CH
                for j in range(CH):
                    wv = plsc.load_gather(
                        w_v, [jnp.full((L,), row0 + j, jnp.int32)]
                    )

                    @pl.loop(0, VPR, unroll=8)
                    def _(g):
                        buf_v[b, j, pl.ds(g * L, L)] = (
                            buf_v[b, j, pl.ds(g * L, L)] * wv
                        )

                scatter_start(c, b)

        for b in range(NBUF):
            scatter_wait(b)

    return k(kv_flat, idx_flat, w_flat)


def kernel(r_idx, r_weight, kv):
    n, m, c_kv = kv.shape
    topk = r_idx.shape[-1]
    R = n * m * topk
    kv_flat = kv.reshape(n * m, c_kv)
    idx_flat = r_idx.reshape(R).astype(jnp.int32)
    w_flat = r_weight.reshape(R)
    out = _gather_scale(kv_flat, idx_flat, w_flat, n=n, m=m, topk=topk, c_kv=c_kv)
    return out.reshape(n, m, topk, c_kv)
